# baseline (speedup 1.0000x reference)
import jax
import jax.numpy as jnp
from jax.experimental import pallas as pl
from jax.experimental.pallas import tpu as pltpu
from jax.experimental.pallas import tpu_sc as plsc

_WINDOW = 512

_TWV = 4096


def _linearize_table(table):
    table_t = table.T
    dim, v = table_t.shape
    nblk = pl.cdiv(v // 2, _TWV)
    if (2 * nblk - 1) * _TWV < v:
        nblk += 1
    split = (nblk - 1) * _TWV
    prows = nblk * _TWV

    def body(lo_ref, hi_ref, out_ref):
        out_ref[...] = jnp.concatenate(
            [jnp.transpose(lo_ref[...]), jnp.transpose(hi_ref[...])], axis=1
        )

    packed = pl.pallas_call(
        body,
        grid=(nblk,),
        in_specs=[
            pl.BlockSpec((dim, _TWV), lambda i: (0, i)),
            pl.BlockSpec((dim, _TWV), lambda i, _o=nblk - 1: (0, i + _o)),
        ],
        out_specs=pl.BlockSpec((_TWV, 2 * dim), lambda i: (i, 0)),
        out_shape=jax.ShapeDtypeStruct((prows, 2 * dim), jnp.float32),
        compiler_params=pltpu.CompilerParams(
            dimension_semantics=("parallel",)
        ),
    )(table_t, table_t)
    return packed.reshape(2 * prows, dim), split


_TBB = 1024


def _transpose_out(g, batch, hist, embed_dim):
    kh = hist // 2
    d2 = 2 * embed_dim
    g2 = g.reshape(kh * batch, d2)

    def body(in_ref, out_ref):
        out_ref[0] = jnp.transpose(in_ref[...])

    out_t = pl.pallas_call(
        body,
        grid=(kh, batch // _TBB),
        in_specs=[pl.BlockSpec((_TBB, d2), lambda k, j, _nb=batch // _TBB: (k * _nb + j, 0))],
        out_specs=pl.BlockSpec((1, d2, _TBB), lambda k, j: (k, 0, j)),
        out_shape=jax.ShapeDtypeStruct((kh, d2, batch), jnp.float32),
        compiler_params=pltpu.CompilerParams(
            dimension_semantics=("parallel", "parallel")
        ),
    )(g2)
    return jnp.transpose(out_t.reshape(hist, embed_dim, batch), (2, 0, 1))


def kernel(x, table):
    batch, hist = x.shape
    n = batch * hist
    embed_dim = table.shape[1]
    table, split = _linearize_table(table)
    xt = x.T
    idx2 = jnp.where(xt < split, 2 * xt, 2 * (xt - split) + 1)

    half = _WINDOW // 2
    wins_per_row = batch // half

    mesh = plsc.VectorSubcoreMesh(core_axis_name="c", subcore_axis_name="s")

    @jax.jit
    @pl.kernel(
        out_type=jax.ShapeDtypeStruct((n, embed_dim), table.dtype),
        mesh=mesh,
        scratch_types=[pltpu.VMEM((_WINDOW,), jnp.int32)],
        compiler_params=pltpu.CompilerParams(
            use_tc_tiling_on_sc=False, needs_layout_passes=False
        ),
    )
    def gather_kernel(tab_hbm, idx_hbm, out_hbm, idxw):
        def body(lo_vmem, hi_vmem, out_vmem):
            for c in range(half // 16):
                slots = jax.lax.iota(jnp.int32, 16) * 2 + (32 * c)
                plsc.store_scatter(idxw, [slots], lo_vmem[0, pl.ds(c * 16, 16)])
                plsc.store_scatter(idxw, [slots + 1], hi_vmem[0, pl.ds(c * 16, 16)])
            pltpu.sync_copy(tab_hbm.at[idxw], out_vmem)

        pltpu.emit_pipeline(
            body,
            grid=(n // _WINDOW,),
            in_specs=[
                pl.BlockSpec((1, half), index_map=lambda w: (2 * (w // wins_per_row), w % wins_per_row)),
                pl.BlockSpec((1, half), index_map=lambda w: (2 * (w // wins_per_row) + 1, w % wins_per_row)),
            ],
            out_specs=[
                pl.BlockSpec((_WINDOW, embed_dim), index_map=lambda w: (w, 0)),
            ],
            core_axis_name=("c", "s"),
            dimension_semantics=(pltpu.PARALLEL,),
        )(idx_hbm, idx_hbm, out_hbm)

    out = gather_kernel(table, idx2)
    return _transpose_out(out, batch, hist, embed_dim)


# speedup vs baseline: 1.4012x; 1.4012x over previous
import jax
import jax.numpy as jnp
from jax.experimental import pallas as pl
from jax.experimental.pallas import tpu as pltpu
from jax.experimental.pallas import tpu_sc as plsc

_WINDOW = 512

_TWV = 8192


def _linearize_table(table):
    table_t = table.T
    dim, v = table_t.shape
    nblk = pl.cdiv(v // 2, _TWV)
    if (2 * nblk - 1) * _TWV < v:
        nblk += 1
    split = (nblk - 1) * _TWV
    prows = nblk * _TWV

    def body(lo_ref, hi_ref, out_ref):
        out_ref[...] = jnp.concatenate(
            [jnp.transpose(lo_ref[...]), jnp.transpose(hi_ref[...])], axis=1
        )

    packed = pl.pallas_call(
        body,
        grid=(nblk,),
        in_specs=[
            pl.BlockSpec((dim, _TWV), lambda i: (0, i)),
            pl.BlockSpec((dim, _TWV), lambda i, _o=nblk - 1: (0, i + _o)),
        ],
        out_specs=pl.BlockSpec((_TWV, 2 * dim), lambda i: (i, 0)),
        out_shape=jax.ShapeDtypeStruct((prows, 2 * dim), jnp.float32),
        compiler_params=pltpu.CompilerParams(
            dimension_semantics=("parallel",)
        ),
    )(table_t, table_t)
    return packed.reshape(2 * prows, dim), split


_TBB = 8192


def _transpose_out(g, batch, hist, embed_dim):
    kh = hist // 2
    d2 = 2 * embed_dim
    g2 = g.reshape(kh * batch, d2)

    def body(in_ref, out_ref):
        out_ref[0] = jnp.transpose(in_ref[...])

    out_t = pl.pallas_call(
        body,
        grid=(kh, batch // _TBB),
        in_specs=[pl.BlockSpec((_TBB, d2), lambda k, j, _nb=batch // _TBB: (k * _nb + j, 0))],
        out_specs=pl.BlockSpec((1, d2, _TBB), lambda k, j: (k, 0, j)),
        out_shape=jax.ShapeDtypeStruct((kh, d2, batch), jnp.float32),
        compiler_params=pltpu.CompilerParams(
            dimension_semantics=("parallel", "parallel")
        ),
    )(g2)
    return jnp.transpose(out_t.reshape(hist, embed_dim, batch), (2, 0, 1))


def kernel(x, table):
    batch, hist = x.shape
    n = batch * hist
    embed_dim = table.shape[1]
    table, split = _linearize_table(table)
    xt = x.T
    idx2 = jnp.where(xt < split, 2 * xt, 2 * (xt - split) + 1)

    half = _WINDOW // 2
    wins_per_row = batch // half

    mesh = plsc.VectorSubcoreMesh(core_axis_name="c", subcore_axis_name="s")

    @jax.jit
    @pl.kernel(
        out_type=jax.ShapeDtypeStruct((n, embed_dim), table.dtype),
        mesh=mesh,
        scratch_types=[pltpu.VMEM((_WINDOW,), jnp.int32)],
        compiler_params=pltpu.CompilerParams(
            use_tc_tiling_on_sc=False, needs_layout_passes=False
        ),
    )
    def gather_kernel(tab_hbm, idx_hbm, out_hbm, idxw):
        def body(lo_vmem, hi_vmem, out_vmem):
            for c in range(half // 16):
                slots = jax.lax.iota(jnp.int32, 16) * 2 + (32 * c)
                plsc.store_scatter(idxw, [slots], lo_vmem[0, pl.ds(c * 16, 16)])
                plsc.store_scatter(idxw, [slots + 1], hi_vmem[0, pl.ds(c * 16, 16)])
            pltpu.sync_copy(tab_hbm.at[idxw], out_vmem)

        pltpu.emit_pipeline(
            body,
            grid=(n // _WINDOW,),
            in_specs=[
                pl.BlockSpec((1, half), index_map=lambda w: (2 * (w // wins_per_row), w % wins_per_row)),
                pl.BlockSpec((1, half), index_map=lambda w: (2 * (w // wins_per_row) + 1, w % wins_per_row)),
            ],
            out_specs=[
                pl.BlockSpec((_WINDOW, embed_dim), index_map=lambda w: (w, 0)),
            ],
            core_axis_name=("c", "s"),
            dimension_semantics=(pltpu.PARALLEL,),
        )(idx_hbm, idx_hbm, out_hbm)

    out = gather_kernel(table, idx2)
    return _transpose_out(out, batch, hist, embed_dim)
